# Initial kernel scaffold; baseline (speedup 1.0000x reference)
import jax
import jax.numpy as jnp
from jax.experimental import pallas as pl


def kernel(X, A_tilde, W):
    raise NotImplementedError("write your pallas kernel here")


# speedup vs baseline: 1.0408x; 1.0408x over previous
import jax
import jax.numpy as jnp
from jax.experimental import pallas as pl
from jax.experimental.pallas import tpu as pltpu

_TM = 400


def _gcn_block(a_ref, x_ref, w_ref, o_ref):
    ax = jnp.dot(a_ref[...], x_ref[...], preferred_element_type=jnp.float32)
    o_ref[...] = jnp.dot(ax, w_ref[...].T, preferred_element_type=jnp.float32)


def kernel(X, A_tilde, W):
    n, d_in = X.shape
    d_out = W.shape[0]
    return pl.pallas_call(
        _gcn_block,
        grid=(n // _TM,),
        in_specs=[
            pl.BlockSpec((_TM, n), lambda i: (i, 0)),
            pl.BlockSpec((n, d_in), lambda i: (0, 0)),
            pl.BlockSpec((d_out, d_in), lambda i: (0, 0)),
        ],
        out_specs=pl.BlockSpec((_TM, d_out), lambda i: (i, 0)),
        out_shape=jax.ShapeDtypeStruct((n, d_out), jnp.float32),
        compiler_params=pltpu.CompilerParams(dimension_semantics=("parallel",)),
    )(A_tilde, X, W)
